# baseline (speedup 1.0000x reference)
import functools

import jax
import jax.numpy as jnp
from jax import lax
from jax.experimental import pallas as pl
from jax.experimental.pallas import tpu as pltpu
from jax.experimental.pallas import tpu_sc as plsc

_MARGIN = 0.5
_ALPHA = 0.01
_BATCHSZ = 1024.0
_K = 3
_C = 28
_CP = 32
_NS = 16
_RT = 1024 // _NS


def _prep_kernel(x_ref, t_ref, vpos_ref, vneg_ref, par_ref):
    x = x_ref[:, :]
    t = t_ref[:, :]
    R, C = x.shape

    bce = jnp.sum(jnp.maximum(x, 0.0) - x * t
                  + jnp.log1p(jnp.exp(-jnp.abs(x)))) / (R * C)

    sig = jax.nn.sigmoid(x)
    pos = t == 1.0
    pad = jnp.full((R, _CP - _C), jnp.inf, jnp.float32)
    vpos_ref[:, :] = jnp.concatenate([jnp.where(pos, sig, jnp.inf), pad],
                                     axis=1)
    vneg_ref[:, :] = jnp.concatenate([jnp.where(pos, jnp.inf, sig), pad],
                                     axis=1)

    counts_col = jax.lax.dot_general(
        t, jnp.ones((R, 1), jnp.float32), (((0,), (0,)), ((), ())),
        preferred_element_type=jnp.float32)
    counts_row = jnp.sum(t, axis=0, keepdims=True)

    k_idx = jax.lax.broadcasted_iota(jnp.int32, (C, C), 0)
    j_idx = jax.lax.broadcasted_iota(jnp.int32, (C, C), 1)
    le = (counts_col < counts_row) | (
        (counts_col == counts_row) & (k_idx <= j_idx))
    prefix = jnp.sum(jnp.where(le, counts_col, 0.0), axis=0, keepdims=True)
    sel = (prefix <= 0.5 * _BATCHSZ) & (counts_row > 1.0)

    n_p = jnp.minimum(jnp.float32(_K), counts_row - 1.0)
    n_n = jnp.minimum(jnp.float32(_K), R - counts_row)
    gate = (sel & (n_p > 0.0) & (n_n > 0.0)).astype(jnp.float32)
    has_any = jnp.max(gate)

    zpad = jnp.zeros((1, _CP - _C), jnp.float32)
    rows = [
        jnp.concatenate([n_p, zpad], axis=1),
        jnp.concatenate([n_n, zpad], axis=1),
        jnp.concatenate([gate, zpad], axis=1),
        jnp.full((1, _CP), bce, jnp.float32),
        jnp.full((1, _CP), has_any, jnp.float32),
        jnp.zeros((3, _CP), jnp.float32),
    ]
    par_ref[:, :] = jnp.concatenate(rows, axis=0)


def _bubble_insert(cur, x):
    for k in range(len(cur)):
        lo = jnp.minimum(cur[k], x)
        x = jnp.maximum(cur[k], x)
        cur[k] = lo
    return cur


def _sc_body(vpos_hbm, vneg_hbm, par_hbm, out_hbm,
             vp_v, vn_v, par_v, cand_v, glob_v, rd_v, part_v, out_v,
             sh_cand, sh_glob, sh_part):
    s = lax.axis_index("s")
    c = lax.axis_index("c")
    pltpu.sync_copy(vpos_hbm.at[pl.ds(s * _RT, _RT)], vp_v)
    pltpu.sync_copy(vneg_hbm.at[pl.ds(s * _RT, _RT)], vn_v)
    pltpu.sync_copy(par_hbm, par_v)

    inf16 = jnp.full((16,), jnp.inf, jnp.float32)

    def row_step(r, carry):
        cur = [list(carry[2 * k:2 * k + 2]) for k in range(_K + 1 + _K)]
        curp, curn = cur[:_K + 1], cur[_K + 1:]
        for h in range(2):
            xv = vp_v[r, pl.ds(16 * h, 16)]
            col = [curp[k][h] for k in range(_K + 1)]
            col = _bubble_insert(col, xv)
            for k in range(_K + 1):
                curp[k][h] = col[k]
            yv = vn_v[r, pl.ds(16 * h, 16)]
            col = [curn[k][h] for k in range(_K)]
            col = _bubble_insert(col, yv)
            for k in range(_K):
                curn[k][h] = col[k]
        return tuple(v for pair in (curp + curn) for v in pair)

    init = tuple(inf16 for _ in range(2 * (2 * _K + 1)))
    fin = lax.fori_loop(0, _RT, row_step, init)
    for k in range(2 * _K + 1):
        for h in range(2):
            cand_v[k, pl.ds(16 * h, 16)] = fin[2 * k + h]
    for h in range(2):
        cand_v[7, pl.ds(16 * h, 16)] = inf16
    pltpu.sync_copy(cand_v, sh_cand.at[pl.ds(s * 8, 8)])
    plsc.subcore_barrier()

    @pl.when(s == 0)
    def _merge():
        gp = [[inf16, inf16] for _ in range(_K + 1)]
        gn = [[inf16, inf16] for _ in range(_K)]
        for w in range(_NS):
            pltpu.sync_copy(sh_cand.at[pl.ds(w * 8, 8)], rd_v)
            for h in range(2):
                for k in range(_K + 1):
                    col = [gp[i][h] for i in range(_K + 1)]
                    col = _bubble_insert(col, rd_v[k, pl.ds(16 * h, 16)])
                    for i in range(_K + 1):
                        gp[i][h] = col[i]
                for k in range(_K):
                    col = [gn[i][h] for i in range(_K)]
                    col = _bubble_insert(col, rd_v[_K + 1 + k,
                                                   pl.ds(16 * h, 16)])
                    for i in range(_K):
                        gn[i][h] = col[i]
        for k in range(_K + 1):
            for h in range(2):
                glob_v[k, pl.ds(16 * h, 16)] = gp[k][h]
        for k in range(_K):
            for h in range(2):
                glob_v[_K + 1 + k, pl.ds(16 * h, 16)] = gn[k][h]
        for h in range(2):
            glob_v[7, pl.ds(16 * h, 16)] = inf16
        pltpu.sync_copy(glob_v, sh_glob)
    plsc.subcore_barrier()
    pltpu.sync_copy(sh_glob, glob_v)

    halves = []
    for h in range(2):
        ds16 = pl.ds(16 * h, 16)
        n_p = par_v[0, ds16]
        n_n = par_v[1, ds16]
        g = par_v[2, ds16]
        sv = [glob_v[k, ds16] for k in range(_K + 1)]
        uv = [glob_v[_K + 1 + k, ds16] for k in range(_K)]
        tf = jnp.minimum(jnp.maximum(n_p, 0.0), jnp.float32(_K))
        s_t = jnp.where(tf == 0.0, sv[0],
                        jnp.where(tf == 1.0, sv[1],
                                  jnp.where(tf == 2.0, sv[2], sv[3])))
        halves.append((n_p, n_n, g * n_n, g * n_p, sv, uv, s_t))

    def acc_step(r, carry):
        acc = list(carry)
        for h in range(2):
            n_p, n_n, w_p, w_n, sv, uv, s_t = halves[h]
            a = vp_v[r, pl.ds(16 * h, 16)]
            fin_m = a < 2.0
            L = n_p + jnp.where(a <= s_t, 1.0, 0.0)
            sp = jnp.zeros((16,), jnp.float32)
            for i in range(_K + 1):
                m = fin_m & (jnp.float32(i) < L)
                sp = sp + jnp.where(m, jnp.abs(a - sv[i]), 0.0)
            sn = jnp.zeros((16,), jnp.float32)
            for i in range(_K):
                m = fin_m & (jnp.float32(i) < n_n)
                sn = sn + jnp.where(m, jnp.abs(a - uv[i]), 0.0)
            acc[h] = acc[h] + (w_p * sp - w_n * sn)
        return tuple(acc)

    z16 = jnp.zeros((16,), jnp.float32)
    acc0, acc1 = lax.fori_loop(0, _RT, acc_step, (z16, z16))
    part_v[0, pl.ds(0, 16)] = acc0 + acc1
    pltpu.sync_copy(part_v, sh_part.at[pl.ds(s, 1)])
    plsc.subcore_barrier()

    tot = jnp.zeros((16,), jnp.float32)
    for w in range(_NS):
        pltpu.sync_copy(sh_part.at[pl.ds(w, 1)], part_v)
        tot = tot + part_v[0, pl.ds(0, 16)]
    lanes = lax.iota(jnp.int32, 16)
    dnums = lax.GatherDimensionNumbers(
        offset_dims=(), collapsed_slice_dims=(0,), start_index_map=(0,))
    rv = tot
    for k in (8, 4, 2, 1):
        rv = rv + lax.gather(
            rv, (lanes ^ k)[:, None], dnums, slice_sizes=(1,),
            mode=lax.GatherScatterMode.PROMISE_IN_BOUNDS)
    bce_v = par_v[3, pl.ds(0, 16)]
    has_v = par_v[4, pl.ds(0, 16)]
    crl_v = jnp.maximum(rv + _MARGIN, 0.0)
    res_v = jnp.where(has_v > 0.0,
                      _ALPHA * crl_v + (1.0 - _ALPHA) * bce_v, bce_v)
    out_v[pl.ds(0, 16)] = res_v

    @pl.when(jnp.logical_and(s == 0, c == 0))
    def _finish():
        pltpu.sync_copy(out_v, out_hbm)


_sc_kernel = functools.partial(
    pl.kernel,
    out_type=jax.ShapeDtypeStruct((16,), jnp.float32),
    mesh=plsc.VectorSubcoreMesh(core_axis_name="c", subcore_axis_name="s"),
    scratch_types=[
        pltpu.VMEM((_RT, _CP), jnp.float32),
        pltpu.VMEM((_RT, _CP), jnp.float32),
        pltpu.VMEM((8, _CP), jnp.float32),
        pltpu.VMEM((8, _CP), jnp.float32),
        pltpu.VMEM((8, _CP), jnp.float32),
        pltpu.VMEM((8, _CP), jnp.float32),
        pltpu.VMEM((1, 16), jnp.float32),
        pltpu.VMEM((16,), jnp.float32),
        pltpu.VMEM_SHARED((8 * _NS, _CP), jnp.float32),
        pltpu.VMEM_SHARED((8, _CP), jnp.float32),
        pltpu.VMEM_SHARED((_NS, 16), jnp.float32),
    ],
)(_sc_body)


def kernel(input, target, X):
    del X
    vpos, vneg, par = pl.pallas_call(
        _prep_kernel,
        out_shape=[
            jax.ShapeDtypeStruct((1024, _CP), jnp.float32),
            jax.ShapeDtypeStruct((1024, _CP), jnp.float32),
            jax.ShapeDtypeStruct((8, _CP), jnp.float32),
        ],
    )(input, target)
    out = _sc_kernel(vpos, vneg, par)
    return out[0]


# speedup vs baseline: 5.6935x; 5.6935x over previous
import jax
import jax.numpy as jnp
from jax.experimental import pallas as pl
from jax.experimental.pallas import tpu as pltpu

_MARGIN = 0.5
_ALPHA = 0.01
_BATCHSZ = 1024.0
_K = 3
_G = 4
_C = 28


def _extract_smallest(v, iota, n):
    outs = []
    for i in range(n):
        m = jnp.min(v, axis=0, keepdims=True)
        outs.append(m)
        if i < n - 1:
            amin = jnp.min(jnp.where(v == m, iota, jnp.int32(1 << 30)),
                           axis=0, keepdims=True)
            v = jnp.where(iota == amin, jnp.inf, v)
    return outs


def _merge_chunks(rows_112):
    parts = []
    for r in rows_112:
        for g in range(_G):
            parts.append(r[:, g * _C:(g + 1) * _C])
    return jnp.concatenate(parts, axis=0)


def _tile_lanes(x):
    return jnp.concatenate([x] * _G, axis=1)


def _crl_kernel(xp_ref, tp_ref, out_ref):
    xp = xp_ref[:, :]
    tp = tp_ref[:, :]
    Rp, W = xp.shape
    R = Rp * _G

    sig = jax.nn.sigmoid(xp)

    nonneg = xp >= 0.0
    bce = jnp.sum(jnp.maximum(xp, 0.0) - xp * tp
                  - jnp.log(jnp.where(nonneg, sig, 1.0 - sig))) / (R * _C)

    per_lane_col = jax.lax.dot_general(
        tp, jnp.ones((Rp, 1), jnp.float32), (((0,), (0,)), ((), ())),
        preferred_element_type=jnp.float32)
    et = (jax.lax.broadcasted_iota(jnp.int32, (_C, W), 1) % _C
          == jax.lax.broadcasted_iota(jnp.int32, (_C, W), 0)
          ).astype(jnp.float32)
    counts_col = jax.lax.dot_general(
        et, per_lane_col, (((1,), (0,)), ((), ())),
        preferred_element_type=jnp.float32)
    per_lane_row = jnp.sum(tp, axis=0, keepdims=True)
    e = (jax.lax.broadcasted_iota(jnp.int32, (W, _C), 0) % _C
         == jax.lax.broadcasted_iota(jnp.int32, (W, _C), 1)
         ).astype(jnp.float32)
    counts_row = jax.lax.dot_general(
        per_lane_row, e, (((1,), (0,)), ((), ())),
        preferred_element_type=jnp.float32)

    k_idx = jax.lax.broadcasted_iota(jnp.int32, (_C, _C), 0)
    j_idx = jax.lax.broadcasted_iota(jnp.int32, (_C, _C), 1)
    le = (counts_col < counts_row) | (
        (counts_col == counts_row) & (k_idx <= j_idx))
    prefix = jnp.sum(jnp.where(le, counts_col, 0.0), axis=0,
                     keepdims=True)
    sel = (prefix <= 0.5 * _BATCHSZ) & (counts_row > 1.0)

    n_p = jnp.minimum(jnp.float32(_K), counts_row - 1.0)
    n_n = jnp.minimum(jnp.float32(_K), R - counts_row)

    pos = tp == 1.0
    riota = jax.lax.broadcasted_iota(jnp.int32, (Rp, W), 0)
    miota_p = jax.lax.broadcasted_iota(jnp.int32, ((_K + 1) * _G, _C), 0)
    miota_n = jax.lax.broadcasted_iota(jnp.int32, (_K * _G, _C), 0)

    s_chunk = _extract_smallest(jnp.where(pos, sig, jnp.inf), riota, _K + 1)
    s_vals = _extract_smallest(_merge_chunks(s_chunk), miota_p, _K + 1)

    u_chunk = _extract_smallest(jnp.where(pos, jnp.inf, sig), riota, _K)
    u_vals = _extract_smallest(_merge_chunks(u_chunk), miota_n, _K)

    t_f = jnp.clip(n_p, 0.0, jnp.float32(_K))
    s_t = jnp.where(t_f == 0.0, s_vals[0],
                    jnp.where(t_f == 1.0, s_vals[1],
                              jnp.where(t_f == 2.0, s_vals[2], s_vals[3])))

    s_t_w = _tile_lanes(s_t)
    n_p_w = _tile_lanes(n_p)
    n_n_w = _tile_lanes(n_n)

    L = n_p_w + (sig <= s_t_w).astype(jnp.float32)
    sum_pos = jnp.zeros_like(sig)
    for i in range(_K + 1):
        sum_pos = sum_pos + jnp.where(
            jnp.float32(i) < L, jnp.abs(sig - _tile_lanes(s_vals[i])), 0.0)
    sum_neg = jnp.zeros_like(sig)
    for i in range(_K):
        sum_neg = sum_neg + jnp.where(
            jnp.float32(i) < n_n_w, jnp.abs(sig - _tile_lanes(u_vals[i])),
            0.0)

    gate = (sel & (n_p > 0.0) & (n_n > 0.0)).astype(jnp.float32)
    vf = pos.astype(jnp.float32) * _tile_lanes(gate)
    d = jnp.sum(vf * (n_n_w * sum_pos - n_p_w * sum_neg))
    has_any = jnp.max(vf) > 0.0

    crl = jnp.maximum(d + _MARGIN, 0.0)
    out_ref[0, 0] = jnp.where(has_any, _ALPHA * crl + (1.0 - _ALPHA) * bce,
                              bce)


def _pack(a):
    return a.reshape(256, _G * _C)


def kernel(input, target, X):
    del X
    out = pl.pallas_call(
        _crl_kernel,
        out_shape=jax.ShapeDtypeStruct((1, 1), jnp.float32),
        out_specs=pl.BlockSpec(memory_space=pltpu.SMEM),
    )(_pack(input), _pack(target))
    return jnp.reshape(out, ())
